# baseline (speedup 1.0000x reference)
import jax
import jax.numpy as jnp
import numpy as np
from jax.experimental import pallas as pl
from jax.experimental.pallas import tpu as pltpu

EPS = np.float32(1e-07)
A = 100000
B = 128
NB = 2
R = 8
TINY = np.float32(np.finfo(np.float32).tiny)


def _threefry_bits(ks0, ks1, cnt):
    rot_a = (13, 15, 26, 6)
    rot_b = (17, 29, 16, 24)
    ks2 = ks0 ^ ks1 ^ np.uint32(0x1BD11BDA)
    ks = (ks0, ks1, ks2)
    x0 = jnp.broadcast_to(ks0, cnt.shape)
    x1 = cnt + ks1
    for i in range(5):
        for r in (rot_a if i % 2 == 0 else rot_b):
            x0 = x0 + x1
            x1 = jax.lax.shift_left(x1, np.uint32(r)) | jax.lax.shift_right_logical(
                x1, np.uint32(32 - r))
            x1 = x0 ^ x1
        x0 = x0 + ks[(i + 1) % 3]
        x1 = x1 + ks[(i + 2) % 3] + np.uint32(i + 1)
    return x0 ^ x1


def _block_kernel(keys_ref, logits_ref, mask_ref, probs_ref, logp_ref, samp_ref):
    r = pl.program_id(0)
    col = jax.lax.broadcasted_iota(jnp.uint32, (R, A), 1)
    rbase = jnp.uint32(R) * r.astype(jnp.uint32)
    row = jax.lax.broadcasted_iota(jnp.uint32, (R, 1), 0) + rbase

    for k in range(NB):
        bits = _threefry_bits(keys_ref[k, 0], keys_ref[k, 1],
                              row * np.uint32(A) + col)
        fb = jax.lax.shift_right_logical(bits, np.uint32(9)) | np.uint32(0x3F800000)
        f = jax.lax.bitcast_convert_type(fb, jnp.float32) - np.float32(1.0)
        g = -jnp.log(-jnp.log(jnp.where(f == 0.0, TINY, f)))

        x = logits_ref[k]
        m = jnp.max(x, axis=1, keepdims=True)
        e = jnp.exp(x - m)
        s = jnp.sum(e, axis=1, keepdims=True)
        raw = (e / s + EPS) * mask_ref[:, k, :]
        denom = jnp.sum(raw, axis=1, keepdims=True)
        p = raw / denom
        probs_ref[:, k, :] = p
        lp = jnp.log(p + EPS)
        logp_ref[:, k, :] = lp

        z = lp + g
        zm = jnp.max(z, axis=1, keepdims=True)
        ci = jax.lax.bitcast_convert_type(col, jnp.int32)
        idx = jnp.min(jnp.where(z == zm, ci, np.int32(A)), axis=1)
        samp_ref[:, k * 128:(k + 1) * 128] = jnp.broadcast_to(
            idx[:, None], (R, 128))


def kernel(branches_logits, action_masks):
    base = jax.random.key(42)
    keys = jnp.stack([jax.random.key_data(jax.random.fold_in(base, k))
                      for k in range(NB)]).astype(jnp.uint32)

    mask3 = action_masks.reshape(B, NB, A)
    probs, logps, samp = pl.pallas_call(
        _block_kernel,
        grid=(B // R,),
        in_specs=[
            pl.BlockSpec(memory_space=pltpu.SMEM),
            pl.BlockSpec((NB, R, A), lambda r: (0, r, 0),
                         pipeline_mode=pl.Buffered(buffer_count=1)),
            pl.BlockSpec((R, NB, A), lambda r: (r, 0, 0)),
        ],
        out_specs=[
            pl.BlockSpec((R, NB, A), lambda r: (r, 0, 0),
                         pipeline_mode=pl.Buffered(buffer_count=1)),
            pl.BlockSpec((R, NB, A), lambda r: (r, 0, 0)),
            pl.BlockSpec((R, NB * 128), lambda r: (r, 0)),
        ],
        out_shape=[
            jax.ShapeDtypeStruct((B, NB, A), jnp.float32),
            jax.ShapeDtypeStruct((B, NB, A), jnp.float32),
            jax.ShapeDtypeStruct((B, NB * 128), jnp.int32),
        ],
        compiler_params=pltpu.CompilerParams(
            dimension_semantics=("parallel",),
            vmem_limit_bytes=112 * 1024 * 1024,
        ),
    )(keys, branches_logits, mask3)

    output = samp[:, ::128]
    return (output, probs.reshape(B, NB * A), logps.reshape(B, NB * A))


# speedup vs baseline: 1.6856x; 1.6856x over previous
import jax
import jax.numpy as jnp
import numpy as np
from jax.experimental import pallas as pl
from jax.experimental.pallas import tpu as pltpu

EPS = np.float32(1e-07)
A = 100000
B = 128
NB = 2
R = 8
TINY = np.float32(np.finfo(np.float32).tiny)


def _threefry_bits(ks0, ks1, cnt):
    rot_a = (13, 15, 26, 6)
    rot_b = (17, 29, 16, 24)
    ks2 = ks0 ^ ks1 ^ np.uint32(0x1BD11BDA)
    ks = (ks0, ks1, ks2)
    x0 = jnp.broadcast_to(ks0, cnt.shape)
    x1 = cnt + ks1
    for i in range(5):
        for r in (rot_a if i % 2 == 0 else rot_b):
            x0 = x0 + x1
            x1 = jax.lax.shift_left(x1, np.uint32(r)) | jax.lax.shift_right_logical(
                x1, np.uint32(32 - r))
            x1 = x0 ^ x1
        x0 = x0 + ks[(i + 1) % 3]
        x1 = x1 + ks[(i + 2) % 3] + np.uint32(i + 1)
    return x0 ^ x1


def _block_kernel(keys_ref, logits_ref, mask_ref, probs_ref, logp_ref, samp_ref):
    r = pl.program_id(0)
    col = jax.lax.broadcasted_iota(jnp.uint32, (R, A), 1)
    rbase = jnp.uint32(R) * r.astype(jnp.uint32)
    row = jax.lax.broadcasted_iota(jnp.uint32, (R, 1), 0) + rbase

    for k in range(NB):
        bits = _threefry_bits(keys_ref[k, 0], keys_ref[k, 1],
                              row * np.uint32(A) + col)
        fb = jax.lax.shift_right_logical(bits, np.uint32(9)) | np.uint32(0x3F800000)
        f = jax.lax.bitcast_convert_type(fb, jnp.float32) - np.float32(1.0)
        g = -jnp.log(-jnp.log(jnp.where(f == 0.0, TINY, f)))

        x = logits_ref[k]
        m = jnp.max(x, axis=1, keepdims=True)
        e = jnp.exp(x - m)
        s = jnp.sum(e, axis=1, keepdims=True)
        raw = (e / s + EPS) * mask_ref[:, k * A:(k + 1) * A]
        denom = jnp.sum(raw, axis=1, keepdims=True)
        p = raw / denom
        probs_ref[:, k * A:(k + 1) * A] = p
        lp = jnp.log(p + EPS)
        logp_ref[:, k * A:(k + 1) * A] = lp

        z = lp + g
        zm = jnp.max(z, axis=1, keepdims=True)
        ci = jax.lax.bitcast_convert_type(col, jnp.int32)
        idx = jnp.min(jnp.where(z == zm, ci, np.int32(A)), axis=1)
        samp_ref[:, k * 128:(k + 1) * 128] = jnp.broadcast_to(
            idx[:, None], (R, 128))


def kernel(branches_logits, action_masks):
    base = jax.random.key(42)
    keys = jnp.stack([jax.random.key_data(jax.random.fold_in(base, k))
                      for k in range(NB)]).astype(jnp.uint32)

    probs, logps, samp = pl.pallas_call(
        _block_kernel,
        grid=(B // R,),
        in_specs=[
            pl.BlockSpec(memory_space=pltpu.SMEM),
            pl.BlockSpec((NB, R, A), lambda r: (0, r, 0),
                         pipeline_mode=pl.Buffered(buffer_count=1)),
            pl.BlockSpec((R, NB * A), lambda r: (r, 0)),
        ],
        out_specs=[
            pl.BlockSpec((R, NB * A), lambda r: (r, 0),
                         pipeline_mode=pl.Buffered(buffer_count=1)),
            pl.BlockSpec((R, NB * A), lambda r: (r, 0)),
            pl.BlockSpec((R, NB * 128), lambda r: (r, 0)),
        ],
        out_shape=[
            jax.ShapeDtypeStruct((B, NB * A), jnp.float32),
            jax.ShapeDtypeStruct((B, NB * A), jnp.float32),
            jax.ShapeDtypeStruct((B, NB * 128), jnp.int32),
        ],
        compiler_params=pltpu.CompilerParams(
            dimension_semantics=("parallel",),
            vmem_limit_bytes=112 * 1024 * 1024,
        ),
    )(keys, branches_logits, action_masks)

    output = samp[:, ::128]
    return (output, probs, logps)


# speedup vs baseline: 1.7039x; 1.0108x over previous
import jax
import jax.numpy as jnp
import numpy as np
from jax.experimental import pallas as pl
from jax.experimental.pallas import tpu as pltpu

EPS = np.float32(1e-07)
A = 100000
B = 128
NB = 2
R = 8
TINY = np.float32(np.finfo(np.float32).tiny)


def _threefry_bits(ks0, ks1, cnt):
    rot_a = (13, 15, 26, 6)
    rot_b = (17, 29, 16, 24)
    ks2 = ks0 ^ ks1 ^ np.uint32(0x1BD11BDA)
    ks = (ks0, ks1, ks2)
    x0 = jnp.broadcast_to(ks0, cnt.shape)
    x1 = cnt + ks1
    for i in range(5):
        for r in (rot_a if i % 2 == 0 else rot_b):
            x0 = x0 + x1
            x1 = jax.lax.shift_left(x1, np.uint32(r)) | jax.lax.shift_right_logical(
                x1, np.uint32(32 - r))
            x1 = x0 ^ x1
        x0 = x0 + ks[(i + 1) % 3]
        x1 = x1 + ks[(i + 2) % 3] + np.uint32(i + 1)
    return x0 ^ x1


def _block_kernel(keys_ref, logits_ref, mask_ref, probs_ref, logp_ref, samp_ref):
    r = pl.program_id(0)
    col = jax.lax.broadcasted_iota(jnp.uint32, (R, A), 1)
    rbase = jnp.uint32(R) * r.astype(jnp.uint32)
    row = jax.lax.broadcasted_iota(jnp.uint32, (R, 1), 0) + rbase

    for k in range(NB):
        bits = _threefry_bits(keys_ref[k, 0], keys_ref[k, 1],
                              row * np.uint32(A) + col)
        fb = jax.lax.shift_right_logical(bits, np.uint32(9)) | np.uint32(0x3F800000)
        f = jax.lax.bitcast_convert_type(fb, jnp.float32) - np.float32(1.0)
        w = -jnp.log(jnp.where(f == 0.0, TINY, f))

        x = logits_ref[k]
        m = jnp.max(x, axis=1, keepdims=True)
        e = jnp.exp(x - m)
        s = jnp.sum(e, axis=1, keepdims=True)
        raw = (e / s + EPS) * mask_ref[:, k * A:(k + 1) * A]
        denom = jnp.sum(raw, axis=1, keepdims=True)
        p = raw / denom
        probs_ref[:, k * A:(k + 1) * A] = p
        t = p + EPS
        logp_ref[:, k * A:(k + 1) * A] = jnp.log(t)

        z = t / w
        zm = jnp.max(z, axis=1, keepdims=True)
        ci = jax.lax.bitcast_convert_type(col, jnp.int32)
        idx = jnp.min(jnp.where(z == zm, ci, np.int32(A)), axis=1)
        samp_ref[:, k * 128:(k + 1) * 128] = jnp.broadcast_to(
            idx[:, None], (R, 128))


def kernel(branches_logits, action_masks):
    base = jax.random.key(42)
    keys = jnp.stack([jax.random.key_data(jax.random.fold_in(base, k))
                      for k in range(NB)]).astype(jnp.uint32)

    probs, logps, samp = pl.pallas_call(
        _block_kernel,
        grid=(B // R,),
        in_specs=[
            pl.BlockSpec(memory_space=pltpu.SMEM),
            pl.BlockSpec((NB, R, A), lambda r: (0, r, 0),
                         pipeline_mode=pl.Buffered(buffer_count=1)),
            pl.BlockSpec((R, NB * A), lambda r: (r, 0)),
        ],
        out_specs=[
            pl.BlockSpec((R, NB * A), lambda r: (r, 0),
                         pipeline_mode=pl.Buffered(buffer_count=1)),
            pl.BlockSpec((R, NB * A), lambda r: (r, 0)),
            pl.BlockSpec((R, NB * 128), lambda r: (r, 0)),
        ],
        out_shape=[
            jax.ShapeDtypeStruct((B, NB * A), jnp.float32),
            jax.ShapeDtypeStruct((B, NB * A), jnp.float32),
            jax.ShapeDtypeStruct((B, NB * 128), jnp.int32),
        ],
        compiler_params=pltpu.CompilerParams(
            dimension_semantics=("parallel",),
            vmem_limit_bytes=112 * 1024 * 1024,
        ),
    )(keys, branches_logits, action_masks)

    output = samp[:, ::128]
    return (output, probs, logps)
